# Initial kernel scaffold; baseline (speedup 1.0000x reference)
import jax
import jax.numpy as jnp
from jax.experimental import pallas as pl


def kernel(inputs, token_table, pos_table):
    raise NotImplementedError("write your pallas kernel here")


# speedup vs baseline: 3.6004x; 3.6004x over previous
import functools

import jax
import jax.numpy as jnp
from jax import lax
from jax.experimental import pallas as pl
from jax.experimental.pallas import tpu as pltpu
from jax.experimental.pallas import tpu_sc as plsc

VOCAB = 100000
DIM = 64
MAXLEN = 200
BATCH = 1024

NC = 2
NS = 16
NW = NC * NS
ROWS_PER_W = BATCH // NW

IDX_SPLIT = 2
IDX_CHUNK = MAXLEN // IDX_SPLIT


def _make_kernel():
  mesh = plsc.VectorSubcoreMesh(core_axis_name="c", subcore_axis_name="s")

  @functools.partial(
      pl.kernel,
      out_type=jax.ShapeDtypeStruct((BATCH, MAXLEN, DIM), jnp.float32),
      mesh=mesh,
      scratch_types=[
          pltpu.VMEM((IDX_SPLIT, IDX_CHUNK), jnp.int32),
          pltpu.VMEM((MAXLEN, DIM), jnp.float32),
          pltpu.SemaphoreType.DMA,
      ],
      compiler_params=pltpu.CompilerParams(use_tc_tiling_on_sc=False),
  )
  def tok_pos_embed(idx_hbm, tok_hbm, pos_hbm, out_hbm, idx_v, row_v, sem):
    wid = lax.axis_index("s") * NC + lax.axis_index("c")

    def body(r, carry):
      row = wid * ROWS_PER_W + r
      pltpu.sync_copy(idx_hbm.at[row], idx_v)
      pltpu.sync_copy(pos_hbm, row_v)
      cps = [
          pltpu.async_copy(
              tok_hbm.at[idx_v.at[j]],
              row_v.at[pl.ds(j * IDX_CHUNK, IDX_CHUNK)],
              sem,
              add=True,
          )
          for j in range(IDX_SPLIT)
      ]
      for cp in cps:
        cp.wait()
      pltpu.sync_copy(row_v, out_hbm.at[row])
      return carry

    lax.fori_loop(0, ROWS_PER_W, body, 0)

  return tok_pos_embed


_KERNEL = _make_kernel()


def kernel(inputs, token_table, pos_table):
  idx = inputs.astype(jnp.int32).reshape(BATCH, IDX_SPLIT, IDX_CHUNK)
  return _KERNEL(idx, token_table, pos_table)
